# baseline (speedup 1.0000x reference)
import functools

import jax
import jax.numpy as jnp
from jax import lax
from jax.experimental import pallas as pl
from jax.experimental.pallas import tpu as pltpu
from jax.experimental.pallas import tpu_sc as plsc

D = 64
DP = 128
L = 16
NC, NS = 2, 16
NW = NC * NS
SCALE = 8.0
NBUF = 4

B = 4096 * 200
BPW = B // NW
CHUNK = 160
NSTEP = BPW // CHUNK

VOCAB = 1000000
BLKV = 4096
NPAIR = (VOCAB + 2 * BLKV - 1) // (2 * BLKV)
VROWS = NPAIR * BLKV

_mesh = plsc.VectorSubcoreMesh(core_axis_name="c", subcore_axis_name="s")


@functools.partial(
    pl.kernel,
    out_type=jax.ShapeDtypeStruct((B, D), jnp.float32),
    mesh=_mesh,
    scratch_types=(
        [pltpu.VMEM((CHUNK,), jnp.int32) for _ in range(NBUF)]
        + [pltpu.VMEM((CHUNK,), jnp.int32) for _ in range(NBUF)]
        + [pltpu.VMEM((CHUNK, DP), jnp.float32) for _ in range(NBUF)]
        + [pltpu.SemaphoreType.DMA for _ in range(NBUF)]
        + [pltpu.SemaphoreType.DMA for _ in range(NBUF)]
        + [pltpu.VMEM((CHUNK, D), jnp.float32) for _ in range(2)]
        + [pltpu.SemaphoreType.DMA for _ in range(NBUF)]
        + [pltpu.SemaphoreType.DMA for _ in range(2)]
    ),
    compiler_params=pltpu.CompilerParams(use_tc_tiling_on_sc=True,
                                        needs_layout_passes=False),
)
def _emb_lookup(gidx_hbm, co_hbm, table_hbm, out_hbm, *bufs):
    gidxb = bufs[0:NBUF]
    cob = bufs[NBUF:2 * NBUF]
    rowsb = bufs[2 * NBUF:3 * NBUF]
    isem = bufs[3 * NBUF:4 * NBUF]
    csem = bufs[4 * NBUF:5 * NBUF]
    outb = bufs[5 * NBUF:5 * NBUF + 2]
    gsem = bufs[5 * NBUF + 2:6 * NBUF + 2]
    osem = bufs[6 * NBUF + 2:6 * NBUF + 4]

    wid = lax.axis_index("s") * NC + lax.axis_index("c")
    base = wid * BPW

    def idx_start(g, j):
        pltpu.async_copy(gidx_hbm.at[pl.ds(base + g * CHUNK, CHUNK)],
                         gidxb[j], isem[j])
        pltpu.async_copy(co_hbm.at[pl.ds(base + g * CHUNK, CHUNK)],
                         cob[j], csem[j])

    def idx_wait(j):
        pltpu.make_async_copy(gidx_hbm.at[pl.ds(base, CHUNK)],
                              gidxb[j], isem[j]).wait()
        pltpu.make_async_copy(co_hbm.at[pl.ds(base, CHUNK)],
                              cob[j], csem[j]).wait()

    def gather_start(j):
        pltpu.async_copy(table_hbm.at[gidxb[j]], rowsb[j], gsem[j])

    def gather_wait(j):
        pltpu.make_async_copy(table_hbm.at[gidxb[j]], rowsb[j], gsem[j]).wait()

    def out_start(g, jo):
        pltpu.async_copy(outb[jo],
                         out_hbm.at[pl.ds(base + g * CHUNK, CHUNK)], osem[jo])

    def out_wait(jo):
        pltpu.make_async_copy(outb[jo],
                              out_hbm.at[pl.ds(base, CHUNK)], osem[jo]).wait()

    for j in range(NBUF):
        idx_start(j, j)
    for j in (0, 1):
        idx_wait(j)
        gather_start(j)

    def chunk_body(g, j, jo):
        gather_wait(j)

        @pl.when(g + 2 < NSTEP)
        def _():
            j2 = (j + 2) % NBUF
            idx_wait(j2)
            gather_start(j2)

        @pl.when(g >= 2)
        def _():
            out_wait(jo)

        iota = lax.iota(jnp.int32, L)

        @plsc.parallel_loop(0, CHUNK // L, 1, unroll=1)
        def _(rg):
            r16 = rg * L + iota
            co16 = cob[j][pl.ds(rg * L, L)]
            wcol = jnp.zeros((L,), jnp.int32)
            for jcol in range(D):
                v = plsc.load_gather(rowsb[j], [r16, co16 + jcol])
                plsc.store_scatter(outb[jo], [r16, wcol + jcol], v)

        out_start(g, jo)

        @pl.when(g + NBUF < NSTEP)
        def _():
            idx_start(g + NBUF, j)

    @pl.loop(0, NSTEP, step=NBUF)
    def _(gg):
        for j in range(NBUF):
            chunk_body(gg + j, j, j % 2)

    out_wait(0)
    out_wait(1)


def _tc_pack_kernel(a_ref, b_ref, out_ref):
    ta = jnp.transpose(a_ref[...] * SCALE, (1, 0))
    tb = jnp.transpose(b_ref[...] * SCALE, (1, 0))
    out_ref[...] = jnp.concatenate([ta, tb], axis=1)


def _tc_format_table(wt):
    return pl.pallas_call(
        _tc_pack_kernel,
        out_shape=jax.ShapeDtypeStruct((VROWS, DP), jnp.float32),
        grid=(NPAIR,),
        in_specs=[pl.BlockSpec((D, BLKV), lambda i: (0, 2 * i)),
                  pl.BlockSpec(
                      (D, BLKV),
                      lambda i: (0, jnp.minimum(2 * i + 1,
                                                (VOCAB - 1) // BLKV)))],
        out_specs=pl.BlockSpec((BLKV, DP), lambda i: (i, 0)),
    )(wt, wt)


def kernel(x, weight):
    wp = _tc_format_table(weight.T)
    xb = x.reshape(B)
    gx = ((xb >> 13) << 12) | (xb & (BLKV - 1))
    co = ((xb >> 12) & 1) << 6
    out = _emb_lookup(gx, co, wp)
    return out.reshape(4096, 200, D)


# speedup vs baseline: 1.9531x; 1.9531x over previous
import functools

import jax
import jax.numpy as jnp
from jax import lax
from jax.experimental import pallas as pl
from jax.experimental.pallas import tpu as pltpu
from jax.experimental.pallas import tpu_sc as plsc

D = 64
DP = 128
L = 16
NC, NS = 2, 16
NW = NC * NS
SCALE = 8.0
NBUF = 4

B = 4096 * 200
BPW = B // NW
CHUNK = 160
NSTEP = BPW // CHUNK

VOCAB = 1000000
BLKV = 4096
NPAIR = (VOCAB + 2 * BLKV - 1) // (2 * BLKV)
VROWS = NPAIR * BLKV

_mesh = plsc.VectorSubcoreMesh(core_axis_name="c", subcore_axis_name="s")


@functools.partial(
    pl.kernel,
    out_type=jax.ShapeDtypeStruct((B, D), jnp.float32),
    mesh=_mesh,
    scratch_types=(
        [pltpu.VMEM((CHUNK,), jnp.int32) for _ in range(NBUF)]
        + [pltpu.VMEM((CHUNK,), jnp.int32) for _ in range(NBUF)]
        + [pltpu.VMEM((CHUNK, DP), jnp.float32) for _ in range(NBUF)]
        + [pltpu.SemaphoreType.DMA for _ in range(NBUF)]
        + [pltpu.SemaphoreType.DMA for _ in range(NBUF)]
        + [pltpu.VMEM((CHUNK, D), jnp.float32) for _ in range(2)]
        + [pltpu.SemaphoreType.DMA for _ in range(NBUF)]
        + [pltpu.SemaphoreType.DMA for _ in range(2)]
    ),
    compiler_params=pltpu.CompilerParams(use_tc_tiling_on_sc=True,
                                        needs_layout_passes=False),
)
def _emb_lookup(gidx_hbm, co_hbm, table_hbm, out_hbm, *bufs):
    gidxb = bufs[0:NBUF]
    cob = bufs[NBUF:2 * NBUF]
    rowsb = bufs[2 * NBUF:3 * NBUF]
    isem = bufs[3 * NBUF:4 * NBUF]
    csem = bufs[4 * NBUF:5 * NBUF]
    outb = bufs[5 * NBUF:5 * NBUF + 2]
    gsem = bufs[5 * NBUF + 2:6 * NBUF + 2]
    osem = bufs[6 * NBUF + 2:6 * NBUF + 4]

    wid = lax.axis_index("s") * NC + lax.axis_index("c")
    base = wid * BPW

    def idx_start(g, j):
        pltpu.async_copy(gidx_hbm.at[pl.ds(base + g * CHUNK, CHUNK)],
                         gidxb[j], isem[j])
        pltpu.async_copy(co_hbm.at[pl.ds(base + g * CHUNK, CHUNK)],
                         cob[j], csem[j])

    def idx_wait(j):
        pltpu.make_async_copy(gidx_hbm.at[pl.ds(base, CHUNK)],
                              gidxb[j], isem[j]).wait()
        pltpu.make_async_copy(co_hbm.at[pl.ds(base, CHUNK)],
                              cob[j], csem[j]).wait()

    def gather_start(j):
        pltpu.async_copy(table_hbm.at[gidxb[j]], rowsb[j], gsem[j])

    def gather_wait(j):
        pltpu.make_async_copy(table_hbm.at[gidxb[j]], rowsb[j], gsem[j]).wait()

    def out_start(g, jo):
        pltpu.async_copy(outb[jo],
                         out_hbm.at[pl.ds(base + g * CHUNK, CHUNK)], osem[jo])

    def out_wait(jo):
        pltpu.make_async_copy(outb[jo],
                              out_hbm.at[pl.ds(base, CHUNK)], osem[jo]).wait()

    for j in range(NBUF):
        idx_start(j, j)
    for j in (0, 1):
        idx_wait(j)
        gather_start(j)

    def chunk_body(g, j, jo):
        gather_wait(j)

        @pl.when(g + 2 < NSTEP)
        def _():
            j2 = (j + 2) % NBUF
            idx_wait(j2)
            gather_start(j2)

        @pl.when(g >= 2)
        def _():
            out_wait(jo)

        @plsc.parallel_loop(0, CHUNK // L, 1, unroll=1)
        def _(rg):
            co16 = cob[j][pl.ds(rg * L, L)]
            for l in range(L):
                r = rg * L + l
                co = co16[l]
                for c in range(D // L):
                    outb[jo][r, pl.ds(c * L, L)] = \
                        rowsb[j][r, pl.ds(co + c * L, L)]

        out_start(g, jo)

        @pl.when(g + NBUF < NSTEP)
        def _():
            idx_start(g + NBUF, j)

    @pl.loop(0, NSTEP, step=NBUF)
    def _(gg):
        for j in range(NBUF):
            chunk_body(gg + j, j, j % 2)

    out_wait(0)
    out_wait(1)


def _tc_pack_kernel(a_ref, b_ref, out_ref):
    ident = (SCALE * jnp.eye(D, dtype=jnp.float32))
    dn = (((0,), (0,)), ((), ()))
    ta = lax.dot_general(a_ref[...], ident, dn,
                         preferred_element_type=jnp.float32)
    tb = lax.dot_general(b_ref[...], ident, dn,
                         preferred_element_type=jnp.float32)
    out_ref[...] = jnp.concatenate([ta, tb], axis=1)


def _tc_format_table(wt):
    return pl.pallas_call(
        _tc_pack_kernel,
        out_shape=jax.ShapeDtypeStruct((VROWS, DP), jnp.float32),
        grid=(NPAIR,),
        in_specs=[pl.BlockSpec((D, BLKV), lambda i: (0, 2 * i)),
                  pl.BlockSpec(
                      (D, BLKV),
                      lambda i: (0, jnp.minimum(2 * i + 1,
                                                (VOCAB - 1) // BLKV)))],
        out_specs=pl.BlockSpec((BLKV, DP), lambda i: (i, 0)),
    )(wt, wt)


def kernel(x, weight):
    wp = _tc_format_table(weight.T)
    xb = x.reshape(B)
    gx = ((xb >> 13) << 12) | (xb & (BLKV - 1))
    co = ((xb >> 12) & 1) << 6
    out = _emb_lookup(gx, co, wp)
    return out.reshape(4096, 200, D)


# speedup vs baseline: 2.1581x; 1.1050x over previous
import functools

import jax
import jax.numpy as jnp
from jax import lax
from jax.experimental import pallas as pl
from jax.experimental.pallas import tpu as pltpu
from jax.experimental.pallas import tpu_sc as plsc

D = 64
DP = 128
L = 16
NC, NS = 2, 16
NW = NC * NS
SCALE = 8.0
NBUF = 4

B = 4096 * 200
BPW = B // NW
CHUNK = 160
NSTEP = BPW // CHUNK

_mesh = plsc.VectorSubcoreMesh(core_axis_name="c", subcore_axis_name="s")


@functools.partial(
    pl.kernel,
    out_type=jax.ShapeDtypeStruct((B, D), jnp.float32),
    mesh=_mesh,
    scratch_types=(
        [pltpu.VMEM((CHUNK,), jnp.int32) for _ in range(NBUF)]
        + [pltpu.VMEM((CHUNK, DP), jnp.float32) for _ in range(NBUF)]
        + [pltpu.VMEM((CHUNK, D), jnp.float32) for _ in range(2)]
        + [pltpu.SemaphoreType.DMA for _ in range(2 * NBUF + 2)]
    ),
    compiler_params=pltpu.CompilerParams(use_tc_tiling_on_sc=True),
)
def _emb_lookup(idx_hbm, table_hbm, out_hbm, *bufs):
    idxb = bufs[0:NBUF]
    rowsb = bufs[NBUF:2 * NBUF]
    outb = bufs[2 * NBUF:2 * NBUF + 2]
    isem = bufs[2 * NBUF + 2:3 * NBUF + 2]
    gsem = bufs[3 * NBUF + 2:4 * NBUF + 2]
    osem = bufs[4 * NBUF + 2:4 * NBUF + 4]

    wid = lax.axis_index("s") * NC + lax.axis_index("c")
    base = wid * BPW

    def idx_start(g, j):
        pltpu.async_copy(idx_hbm.at[pl.ds(base + g * CHUNK, CHUNK)],
                         idxb[j], isem[j])

    def idx_wait(j):
        pltpu.make_async_copy(idx_hbm.at[pl.ds(base, CHUNK)],
                              idxb[j], isem[j]).wait()

    def gather_start(j):
        pltpu.async_copy(table_hbm.at[idxb[j]], rowsb[j], gsem[j])

    def gather_wait(j):
        pltpu.make_async_copy(table_hbm.at[idxb[j]], rowsb[j], gsem[j]).wait()

    def out_start(g, jo):
        pltpu.async_copy(outb[jo],
                         out_hbm.at[pl.ds(base + g * CHUNK, CHUNK)], osem[jo])

    def out_wait(jo):
        pltpu.make_async_copy(outb[jo],
                              out_hbm.at[pl.ds(base, CHUNK)], osem[jo]).wait()

    for j in range(NBUF):
        idx_start(j, j)
    idx_wait(0)
    gather_start(0)
    idx_wait(1)
    gather_start(1)

    def chunk_body(g, j, jo):
        gather_wait(j)

        @pl.when(g + NBUF < NSTEP)
        def _():
            idx_start(g + NBUF, j)

        @pl.when(g + 2 < NSTEP)
        def _():
            j2 = (j + 2) % NBUF
            idx_wait(j2)
            gather_start(j2)

        @pl.when(g >= 2)
        def _():
            out_wait(jo)

        @plsc.parallel_loop(0, CHUNK, 1, unroll=4)
        def _(r):
            for c in range(D // L):
                sl = pl.ds(c * L, L)
                outb[jo][r, sl] = rowsb[j][r, sl]

        out_start(g, jo)

    @pl.loop(0, NSTEP, step=NBUF)
    def _(gg):
        for j in range(NBUF):
            chunk_body(gg + j, j, j % 2)

    out_wait(0)
    out_wait(1)


VOCAB = 1000000
BLKV = 8192


def _tc_transpose_kernel(wt_ref, out_ref):
    out_ref[:, 0:D] = jnp.transpose(wt_ref[...] * SCALE, (1, 0))


def _tc_format_table(wt):
    grid = pl.cdiv(VOCAB, BLKV)
    return pl.pallas_call(
        _tc_transpose_kernel,
        out_shape=jax.ShapeDtypeStruct((VOCAB, DP), jnp.float32),
        grid=(grid,),
        in_specs=[pl.BlockSpec((D, BLKV), lambda i: (0, i))],
        out_specs=pl.BlockSpec((BLKV, DP), lambda i: (i, 0)),
    )(wt)


def kernel(x, weight):
    wp = _tc_format_table(weight.T)
    out = _emb_lookup(x.reshape(B), wp)
    return out.reshape(4096, 200, D)
